# initial kernel scaffold (unmeasured)
import jax
import jax.numpy as jnp
from jax import lax
from jax.experimental import pallas as pl
from jax.experimental.pallas import tpu as pltpu


def kernel(
    x,
):
    def body(*refs):
        pass

    out_shape = jax.ShapeDtypeStruct(..., jnp.float32)
    return pl.pallas_call(body, out_shape=out_shape)(...)


# baseline (device time: 194032 ns/iter reference)
import jax
import jax.numpy as jnp
from jax import lax
from jax.experimental import pallas as pl
from jax.experimental.pallas import tpu as pltpu

N_DEV = 8
N_HOPS = N_DEV - 1


def kernel(x):
    m_per, n = x.shape
    ch = m_per // N_DEV

    def body(x_ref, out_ref, recv_buf, send_buf, send_sem, recv_sems):
        my = lax.axis_index("i")
        left = lax.rem(my + N_DEV - 1, N_DEV)
        right = lax.rem(my + 1, N_DEV)

        barrier_sem = pltpu.get_barrier_semaphore()
        for nbr in (left, right):
            pl.semaphore_signal(
                barrier_sem, inc=1,
                device_id=(nbr,), device_id_type=pl.DeviceIdType.MESH,
            )
        pl.semaphore_wait(barrier_sem, 2)

        send_buf[...] = x_ref[pl.ds(my * ch, ch), :]
        for h in range(N_HOPS):
            rdma = pltpu.make_async_remote_copy(
                src_ref=send_buf,
                dst_ref=recv_buf.at[h],
                send_sem=send_sem,
                recv_sem=recv_sems.at[h],
                device_id=(right,),
                device_id_type=pl.DeviceIdType.MESH,
            )
            rdma.start()
            rdma.wait()
            c = lax.rem(my - h - 1 + 2 * N_DEV, N_DEV)
            acc = recv_buf[h] + x_ref[pl.ds(c * ch, ch), :]
            if h < N_HOPS - 1:
                send_buf[...] = acc
            else:
                out_ref[pl.ds(c * ch, ch), :] = acc
                send_buf[...] = acc

        for g in range(N_HOPS):
            s = N_HOPS + g
            rdma = pltpu.make_async_remote_copy(
                src_ref=send_buf,
                dst_ref=recv_buf.at[s],
                send_sem=send_sem,
                recv_sem=recv_sems.at[s],
                device_id=(right,),
                device_id_type=pl.DeviceIdType.MESH,
            )
            rdma.start()
            rdma.wait()
            c = lax.rem(my - g + N_DEV, N_DEV)
            out_ref[pl.ds(c * ch, ch), :] = recv_buf[s]
            if g < N_HOPS - 1:
                send_buf[...] = recv_buf[s]

    return pl.pallas_call(
        body,
        out_shape=jax.ShapeDtypeStruct((m_per, n), x.dtype),
        in_specs=[pl.BlockSpec(memory_space=pltpu.VMEM)],
        out_specs=pl.BlockSpec(memory_space=pltpu.VMEM),
        scratch_shapes=[
            pltpu.VMEM((2 * N_HOPS, ch, n), x.dtype),
            pltpu.VMEM((ch, n), x.dtype),
            pltpu.SemaphoreType.DMA,
            pltpu.SemaphoreType.DMA((2 * N_HOPS,)),
        ],
        compiler_params=pltpu.CompilerParams(collective_id=0),
    )(x)


# device time: 75328 ns/iter; 2.5758x vs baseline; 2.5758x over previous
import jax
import jax.numpy as jnp
from jax import lax
from jax.experimental import pallas as pl
from jax.experimental.pallas import tpu as pltpu

PART_OFF = (0, 704, 1408)
PART_ROWS = (704, 704, 640)
MASKS = {"X": 1, "Y": 3, "Z": 4}
ORDERS = (("X", "Y", "Z"), ("Y", "Z", "X"), ("Z", "X", "Y"))


def kernel(x):
    m_per, n = x.shape

    def body(x_ref, out_ref, scr0, scr1, scr2, send_sems, recv_sems):
        scr = (scr0, scr1, scr2)
        my = lax.axis_index("i")
        b0 = my & 1
        b1 = (my >> 1) & 1
        b2 = (my >> 2) & 1
        keep = {"X": b0 ^ b1, "Y": b1, "Z": b2}

        barrier_sem = pltpu.get_barrier_semaphore()
        for mask in MASKS.values():
            pl.semaphore_signal(
                barrier_sem, inc=1,
                device_id=(my ^ mask,), device_id_type=pl.DeviceIdType.MESH,
            )
        pl.semaphore_wait(barrier_sem, 3)

        offs = [jnp.int32(PART_OFF[p]) for p in range(3)]

        def scr_sub(p, k):
            r = PART_ROWS[p]
            sub = (0, r // 2, 3 * r // 4)[k]
            return scr[p].at[pl.ds(sub, r >> (k + 1)), :]

        def rs_rdma(p, k):
            axis = ORDERS[p][k]
            half = PART_ROWS[p] >> (k + 1)
            u = keep[axis]
            send_off = offs[p] + (1 - u) * half
            src = (x_ref if k == 0 else out_ref).at[pl.ds(send_off, half), :]
            return pltpu.make_async_remote_copy(
                src_ref=src,
                dst_ref=scr_sub(p, k),
                send_sem=send_sems.at[p],
                recv_sem=recv_sems.at[p, k],
                device_id=(my ^ MASKS[axis],),
                device_id_type=pl.DeviceIdType.MESH,
            )

        def rs_add(p, k):
            axis = ORDERS[p][k]
            half = PART_ROWS[p] >> (k + 1)
            keep_off = offs[p] + keep[axis] * half
            src = x_ref if k == 0 else out_ref
            out_ref[pl.ds(keep_off, half), :] = (
                src[pl.ds(keep_off, half), :] + scr_sub(p, k)[...]
            )
            offs[p] = keep_off

        def ag_rdma(p, j):
            axis = ORDERS[p][2 - j]
            sz = PART_ROWS[p] >> (3 - j)
            region = out_ref.at[pl.ds(offs[p], sz), :]
            return pltpu.make_async_remote_copy(
                src_ref=region,
                dst_ref=region,
                send_sem=send_sems.at[p],
                recv_sem=recv_sems.at[p, 3 + j],
                device_id=(my ^ MASKS[axis],),
                device_id_type=pl.DeviceIdType.MESH,
            )

        def ag_grow(p, j):
            axis = ORDERS[p][2 - j]
            sz = PART_ROWS[p] >> (3 - j)
            offs[p] = offs[p] - keep[axis] * sz

        rd = {}
        for p in range(3):
            rd[p] = rs_rdma(p, 0)
            rd[p].start()
        for k in (1, 2):
            for p in range(3):
                rd[p].wait()
                rs_add(p, k - 1)
                rd[p] = rs_rdma(p, k)
                rd[p].start()
        for p in range(3):
            rd[p].wait()
            rs_add(p, 2)
            rd[p] = ag_rdma(p, 0)
            rd[p].start()
        for j in (1, 2):
            for p in range(3):
                rd[p].wait()
                ag_grow(p, j - 1)
                rd[p] = ag_rdma(p, j)
                rd[p].start()
        for p in range(3):
            rd[p].wait()

    return pl.pallas_call(
        body,
        out_shape=jax.ShapeDtypeStruct((m_per, n), x.dtype),
        in_specs=[pl.BlockSpec(memory_space=pltpu.VMEM)],
        out_specs=pl.BlockSpec(memory_space=pltpu.VMEM),
        scratch_shapes=[
            pltpu.VMEM((7 * PART_ROWS[0] // 8, n), x.dtype),
            pltpu.VMEM((7 * PART_ROWS[1] // 8, n), x.dtype),
            pltpu.VMEM((7 * PART_ROWS[2] // 8, n), x.dtype),
            pltpu.SemaphoreType.DMA((3,)),
            pltpu.SemaphoreType.DMA((3, 6)),
        ],
        compiler_params=pltpu.CompilerParams(collective_id=0),
    )(x)


# device time: 71691 ns/iter; 2.7065x vs baseline; 1.0507x over previous
import jax
import jax.numpy as jnp
from jax import lax
from jax.experimental import pallas as pl
from jax.experimental.pallas import tpu as pltpu

MASKS = {"X": 1, "Y": 3, "Z": 4}
CHAINS = (
    (0, 384, ("X", "Y", "Z")),
    (384, 320, ("X", "Y", "Z")),
    (704, 384, ("Y", "Z", "X")),
    (1088, 320, ("Y", "Z", "X")),
    (1408, 320, ("Z", "X", "Y")),
    (1728, 320, ("Z", "X", "Y")),
)
NC = len(CHAINS)
SCR_BASE = []
_acc = 0
for _, _r, _ in CHAINS:
    SCR_BASE.append(_acc)
    _acc += 7 * _r // 8
SCR_ROWS = _acc


def kernel(x):
    m_per, n = x.shape

    def body(x_ref, out_ref, scr, send_sems, recv_sems):
        my = lax.axis_index("i")
        b0 = my & 1
        b1 = (my >> 1) & 1
        b2 = (my >> 2) & 1
        keep = {"X": b0 ^ b1, "Y": b1, "Z": b2}

        barrier_sem = pltpu.get_barrier_semaphore()
        for mask in MASKS.values():
            pl.semaphore_signal(
                barrier_sem, inc=1,
                device_id=(my ^ mask,), device_id_type=pl.DeviceIdType.MESH,
            )
        pl.semaphore_wait(barrier_sem, 3)

        offs = [jnp.int32(CHAINS[c][0]) for c in range(NC)]

        def scr_sub(c, k):
            r = CHAINS[c][1]
            sub = SCR_BASE[c] + (0, r // 2, 3 * r // 4)[k]
            return scr.at[pl.ds(sub, r >> (k + 1)), :]

        def rs_rdma(c, k):
            axis = CHAINS[c][2][k]
            half = CHAINS[c][1] >> (k + 1)
            u = keep[axis]
            send_off = offs[c] + (1 - u) * half
            src = (x_ref if k == 0 else out_ref).at[pl.ds(send_off, half), :]
            return pltpu.make_async_remote_copy(
                src_ref=src,
                dst_ref=scr_sub(c, k),
                send_sem=send_sems.at[c],
                recv_sem=recv_sems.at[6 * c + k],
                device_id=(my ^ MASKS[axis],),
                device_id_type=pl.DeviceIdType.MESH,
            )

        def rs_add(c, k):
            axis = CHAINS[c][2][k]
            half = CHAINS[c][1] >> (k + 1)
            keep_off = offs[c] + keep[axis] * half
            src = x_ref if k == 0 else out_ref
            out_ref[pl.ds(keep_off, half), :] = (
                src[pl.ds(keep_off, half), :] + scr_sub(c, k)[...]
            )
            offs[c] = keep_off

        def ag_rdma(c, j):
            axis = CHAINS[c][2][2 - j]
            sz = CHAINS[c][1] >> (3 - j)
            region = out_ref.at[pl.ds(offs[c], sz), :]
            return pltpu.make_async_remote_copy(
                src_ref=region,
                dst_ref=region,
                send_sem=send_sems.at[c],
                recv_sem=recv_sems.at[6 * c + 3 + j],
                device_id=(my ^ MASKS[axis],),
                device_id_type=pl.DeviceIdType.MESH,
            )

        def ag_grow(c, j):
            axis = CHAINS[c][2][2 - j]
            sz = CHAINS[c][1] >> (3 - j)
            offs[c] = offs[c] - keep[axis] * sz

        rd = {}
        for c in range(NC):
            rd[c] = rs_rdma(c, 0)
            rd[c].start()
        for s in range(1, 6):
            for c in range(NC):
                rd[c].wait()
                if s <= 2:
                    rs_add(c, s - 1)
                    rd[c] = rs_rdma(c, s)
                elif s == 3:
                    rs_add(c, 2)
                    rd[c] = ag_rdma(c, 0)
                else:
                    ag_grow(c, s - 4)
                    rd[c] = ag_rdma(c, s - 3)
                rd[c].start()
        for c in range(NC):
            rd[c].wait()

    return pl.pallas_call(
        body,
        out_shape=jax.ShapeDtypeStruct((m_per, n), x.dtype),
        in_specs=[pl.BlockSpec(memory_space=pltpu.VMEM)],
        out_specs=pl.BlockSpec(memory_space=pltpu.VMEM),
        scratch_shapes=[
            pltpu.VMEM((SCR_ROWS, n), x.dtype),
            pltpu.SemaphoreType.DMA((NC,)),
            pltpu.SemaphoreType.DMA((6 * NC,)),
        ],
        compiler_params=pltpu.CompilerParams(collective_id=0),
    )(x)


# device time: 65906 ns/iter; 2.9441x vs baseline; 1.0878x over previous
import jax
import jax.numpy as jnp
from jax import lax
from jax.experimental import pallas as pl
from jax.experimental.pallas import tpu as pltpu

MASKS = {"X": 1, "Y": 3, "Z": 4}
CHAINS = (
    (0, 384, ("X", "Y", "Z")),
    (384, 320, ("X", "Y", "Z")),
    (704, 384, ("Y", "Z", "X")),
    (1088, 320, ("Y", "Z", "X")),
    (1408, 320, ("Z", "X", "Y")),
    (1728, 320, ("Z", "X", "Y")),
)
NC = len(CHAINS)
SCR_BASE = []
_acc = 0
for _, _r, _ in CHAINS:
    SCR_BASE.append(_acc)
    _acc += 7 * _r // 8
SCR_ROWS = _acc


def kernel(x):
    m_per, n = x.shape

    def body(x_ref, out_ref, scr, send_sems, recv_sems):
        my = lax.axis_index("i")
        b0 = my & 1
        b1 = (my >> 1) & 1
        b2 = (my >> 2) & 1
        keep = {"X": b0 ^ b1, "Y": b1, "Z": b2}

        barrier_sem = pltpu.get_barrier_semaphore()
        for mask in MASKS.values():
            pl.semaphore_signal(
                barrier_sem, inc=1,
                device_id=(my ^ mask,), device_id_type=pl.DeviceIdType.MESH,
            )
        pl.semaphore_wait(barrier_sem, 3)

        offs = [jnp.int32(CHAINS[c][0]) for c in range(NC)]

        def scr_sub(c, k):
            r = CHAINS[c][1]
            sub = SCR_BASE[c] + (0, r // 2, 3 * r // 4)[k]
            return scr.at[pl.ds(sub, r >> (k + 1)), :]

        def rs_rdma(c, k):
            axis = CHAINS[c][2][k]
            half = CHAINS[c][1] >> (k + 1)
            u = keep[axis]
            send_off = offs[c] + (1 - u) * half
            src = (x_ref if k == 0 else out_ref).at[pl.ds(send_off, half), :]
            return pltpu.make_async_remote_copy(
                src_ref=src,
                dst_ref=scr_sub(c, k),
                send_sem=send_sems.at[c],
                recv_sem=recv_sems.at[6 * c + k],
                device_id=(my ^ MASKS[axis],),
                device_id_type=pl.DeviceIdType.MESH,
            )

        def rs_add(c, k):
            axis = CHAINS[c][2][k]
            half = CHAINS[c][1] >> (k + 1)
            keep_off = offs[c] + keep[axis] * half
            src = x_ref if k == 0 else out_ref
            out_ref[pl.ds(keep_off, half), :] = (
                src[pl.ds(keep_off, half), :] + scr_sub(c, k)[...]
            )
            offs[c] = keep_off

        def rs_add_critical(c, k):
            axis = CHAINS[c][2][k]
            half = CHAINS[c][1] >> (k + 1)
            q = half // 2
            keep_off = offs[c] + keep[axis] * half
            u2 = keep[CHAINS[c][2][k + 1]]
            src = x_ref if k == 0 else out_ref
            r = CHAINS[c][1]
            sub = SCR_BASE[c] + (0, r // 2, 3 * r // 4)[k]

            def piece(rel):
                out_ref[pl.ds(keep_off + rel, q), :] = (
                    src[pl.ds(keep_off + rel, q), :]
                    + scr[pl.ds(sub + rel, q), :]
                )

            piece((1 - u2) * q)
            offs[c] = keep_off
            return lambda: piece(u2 * q)

        def ag_rdma(c, j):
            axis = CHAINS[c][2][2 - j]
            sz = CHAINS[c][1] >> (3 - j)
            region = out_ref.at[pl.ds(offs[c], sz), :]
            return pltpu.make_async_remote_copy(
                src_ref=region,
                dst_ref=region,
                send_sem=send_sems.at[c],
                recv_sem=recv_sems.at[6 * c + 3 + j],
                device_id=(my ^ MASKS[axis],),
                device_id_type=pl.DeviceIdType.MESH,
            )

        def ag_grow(c, j):
            axis = CHAINS[c][2][2 - j]
            sz = CHAINS[c][1] >> (3 - j)
            offs[c] = offs[c] - keep[axis] * sz

        order = (0, 2, 4, 1, 3, 5)
        rd = {}
        for c in order:
            rd[c] = rs_rdma(c, 0)
            rd[c].start()
        for s in range(1, 6):
            for c in order:
                rd[c].wait()
                if s <= 2:
                    lazy = rs_add_critical(c, s - 1)
                    rd[c] = rs_rdma(c, s)
                    rd[c].start()
                    lazy()
                    continue
                elif s == 3:
                    rs_add(c, 2)
                    rd[c] = ag_rdma(c, 0)
                else:
                    ag_grow(c, s - 4)
                    rd[c] = ag_rdma(c, s - 3)
                rd[c].start()
        for c in order:
            rd[c].wait()

    return pl.pallas_call(
        body,
        out_shape=jax.ShapeDtypeStruct((m_per, n), x.dtype),
        in_specs=[pl.BlockSpec(memory_space=pltpu.VMEM)],
        out_specs=pl.BlockSpec(memory_space=pltpu.VMEM),
        scratch_shapes=[
            pltpu.VMEM((SCR_ROWS, n), x.dtype),
            pltpu.SemaphoreType.DMA((NC,)),
            pltpu.SemaphoreType.DMA((6 * NC,)),
        ],
        compiler_params=pltpu.CompilerParams(collective_id=0),
    )(x)
